# Initial kernel scaffold; baseline (speedup 1.0000x reference)
import jax
import jax.numpy as jnp
from jax.experimental import pallas as pl


def kernel(in_feat, edge_index, W0, b0, W1, b1, W2, b2):
    raise NotImplementedError("write your pallas kernel here")


# speedup vs baseline: 4.8276x; 4.8276x over previous
import functools

import jax
import jax.numpy as jnp
import numpy as np
from jax import lax
from jax.experimental import pallas as pl
from jax.experimental.pallas import tpu as pltpu
from jax.experimental.pallas import tpu_sc as plsc

N = 10000
NW = 32
NSUB = 16
CHUNK = 128
CPW = 79
E_PAD = NW * CPW * CHUNK
N1 = 10112
RPT = N1 // NSUB


def _prep_body(src_ref, dst_ref, out_ref):
    s = src_ref[...]
    d = dst_ref[...]
    out_ref[...] = jnp.where(s == d, N, d)


def _prep(srcm, dstm):
    rows = srcm.shape[0]
    return pl.pallas_call(
        _prep_body,
        out_shape=jax.ShapeDtypeStruct((rows, CHUNK), jnp.int32),
    )(srcm, dstm)


def _mm_body(x_ref, w_ref, o_ref):
    o_ref[...] = lax.dot_general(
        x_ref[...], w_ref[...], (((1,), (1,)), ((), ())),
        preferred_element_type=jnp.float32)


def _mm(x, w):
    fo = w.shape[0]
    blk = 2000
    return pl.pallas_call(
        _mm_body,
        grid=(N // blk,),
        in_specs=[
            pl.BlockSpec((blk, x.shape[1]), lambda i: (i, np.int32(0))),
            pl.BlockSpec(w.shape, lambda i: (np.int32(0), np.int32(0))),
        ],
        out_specs=pl.BlockSpec((blk, fo), lambda i: (i, np.int32(0))),
        out_shape=jax.ShapeDtypeStruct((N, fo), jnp.float32),
    )(x, w)


def _ep_mm_body(sa_ref, sb_ref, g_ref, da_ref, db_ref, b_ref, w_ref, o_ref):
    r = 1.0 / (da_ref[...] + db_ref[...] + 2.0)
    h = (sa_ref[...] + sb_ref[...] + 2.0 * g_ref[...]) * r + b_ref[...]
    h = jnp.maximum(h, 0.0)
    y = lax.dot_general(
        h, w_ref[...], (((1,), (1,)), ((), ())),
        preferred_element_type=jnp.float32)
    pad = o_ref.shape[1] - y.shape[1]
    if pad:
        y = jnp.concatenate([y, jnp.zeros((y.shape[0], pad), y.dtype)], axis=1)
    o_ref[...] = y


def _ep_mm(sa, sb, g, da, db, b, w, pad_to=None):
    fi = g.shape[1]
    fo = pad_to if pad_to is not None else w.shape[0]
    blk = 2000
    fspec = pl.BlockSpec((blk, fi), lambda i: (i, np.int32(0)))
    dspec = pl.BlockSpec((blk, 1), lambda i: (i, np.int32(0)))
    return pl.pallas_call(
        _ep_mm_body,
        grid=(N // blk,),
        in_specs=[
            fspec, fspec, fspec, dspec, dspec,
            pl.BlockSpec((1, fi), lambda i: (np.int32(0), np.int32(0))),
            pl.BlockSpec(w.shape, lambda i: (np.int32(0), np.int32(0))),
        ],
        out_specs=pl.BlockSpec((blk, fo), lambda i: (i, np.int32(0))),
        out_shape=jax.ShapeDtypeStruct((N, fo), jnp.float32),
    )(sa, sb, g, da, db, b, w)


def _ep_body(sa_ref, sb_ref, g_ref, da_ref, db_ref, b_ref, o_ref):
    r = 1.0 / (da_ref[...] + db_ref[...] + 2.0)
    o_ref[...] = (sa_ref[...] + sb_ref[...] + 2.0 * g_ref[...]) * r + b_ref[...]


def _ep(sa, sb, g, da, db, b):
    fo = g.shape[1]
    blk = 2000
    fspec = pl.BlockSpec((blk, fo), lambda i: (i, np.int32(0)))
    dspec = pl.BlockSpec((blk, 1), lambda i: (i, np.int32(0)))
    return pl.pallas_call(
        _ep_body,
        grid=(N // blk,),
        in_specs=[
            fspec, fspec, fspec, dspec, dspec,
            pl.BlockSpec((1, fo), lambda i: (np.int32(0), np.int32(0))),
        ],
        out_specs=fspec,
        out_shape=jax.ShapeDtypeStruct((N, fo), jnp.float32),
    )(sa, sb, g, da, db, b)


def _make_agg_deg(D):
    mesh = plsc.VectorSubcoreMesh(core_axis_name="c", subcore_axis_name="s")

    @functools.partial(
        pl.kernel, mesh=mesh,
        out_type=[
            jax.ShapeDtypeStruct((2 * N1, D), jnp.float32),
            jax.ShapeDtypeStruct((2 * N1,), jnp.float32),
        ],
        scratch_types=[
            pltpu.VMEM((CPW, CHUNK), jnp.int32),
            pltpu.VMEM((CPW, CHUNK), jnp.int32),
            pltpu.VMEM((CHUNK, D), jnp.float32),
            pltpu.VMEM((CHUNK,), jnp.float32),
            pltpu.VMEM((RPT,), jnp.float32),
            pltpu.VMEM_SHARED((N1, D), jnp.float32),
            pltpu.VMEM_SHARED((N1,), jnp.float32),
            pltpu.SemaphoreType.DMA,
        ],
    )
    def agg(g_hbm, srcw, dstw, z2, z1, ones_h, out_s, out_d,
            srcv, dstv, buf, onesv, dbuf, acc_s, acc_d, sem):
        cid = lax.axis_index("c")
        sid = lax.axis_index("s")
        wid = cid * NSUB + sid
        row0 = sid * RPT
        pltpu.sync_copy(z2, acc_s.at[pl.ds(row0, RPT)])
        pltpu.sync_copy(z1, dbuf)
        pltpu.sync_copy(dbuf, acc_d.at[pl.ds(row0, RPT)])
        pltpu.sync_copy(srcw.at[wid], srcv)
        pltpu.sync_copy(dstw.at[wid], dstv)
        pltpu.sync_copy(ones_h, onesv)
        plsc.subcore_barrier()

        def step(j, carry):
            pltpu.async_copy(g_hbm.at[srcv.at[j]], buf, sem).wait()
            pltpu.sync_copy(buf, acc_s.at[dstv.at[j]], add=True)
            pltpu.sync_copy(onesv, acc_d.at[dstv.at[j]], add=True)
            return carry

        lax.fori_loop(0, CPW, step, 0)
        plsc.subcore_barrier()
        obase = cid * N1 + row0
        pltpu.sync_copy(acc_s.at[pl.ds(row0, RPT)], out_s.at[pl.ds(obase, RPT)])
        pltpu.sync_copy(acc_d.at[pl.ds(row0, RPT)], dbuf)
        pltpu.sync_copy(dbuf, out_d.at[pl.ds(obase, RPT)])

    return agg


def _make_agg(D):
    mesh = plsc.VectorSubcoreMesh(core_axis_name="c", subcore_axis_name="s")

    @functools.partial(
        pl.kernel, mesh=mesh,
        out_type=jax.ShapeDtypeStruct((2 * N1, D), jnp.float32),
        scratch_types=[
            pltpu.VMEM((CPW, CHUNK), jnp.int32),
            pltpu.VMEM((CPW, CHUNK), jnp.int32),
            pltpu.VMEM((CHUNK, D), jnp.float32),
            pltpu.VMEM_SHARED((N1, D), jnp.float32),
            pltpu.SemaphoreType.DMA,
        ],
    )
    def agg(g_hbm, srcw, dstw, z2, out_s, srcv, dstv, buf, acc_s, sem):
        cid = lax.axis_index("c")
        sid = lax.axis_index("s")
        wid = cid * NSUB + sid
        row0 = sid * RPT
        pltpu.sync_copy(z2, acc_s.at[pl.ds(row0, RPT)])
        pltpu.sync_copy(srcw.at[wid], srcv)
        pltpu.sync_copy(dstw.at[wid], dstv)
        plsc.subcore_barrier()

        def step(j, carry):
            pltpu.async_copy(g_hbm.at[srcv.at[j]], buf, sem).wait()
            pltpu.sync_copy(buf, acc_s.at[dstv.at[j]], add=True)
            return carry

        lax.fori_loop(0, CPW, step, 0)
        plsc.subcore_barrier()
        obase = cid * N1 + row0
        pltpu.sync_copy(acc_s.at[pl.ds(row0, RPT)], out_s.at[pl.ds(obase, RPT)])

    return agg


_make_agg_deg = functools.lru_cache(None)(_make_agg_deg)
_make_agg = functools.lru_cache(None)(_make_agg)


def _agg_deg_128(*args):
    return _make_agg_deg(128)(*args)


def _agg_128(*args):
    return _make_agg(128)(*args)


def kernel(in_feat, edge_index, W0, b0, W1, b1, W2, b2):
    x = in_feat.astype(jnp.float32)
    src = edge_index[0].astype(jnp.int32)
    dst = edge_index[1].astype(jnp.int32)
    e = src.shape[0]
    pad = E_PAD - e
    srcp = jnp.concatenate([src, jnp.zeros((pad,), jnp.int32)])
    dstp = jnp.concatenate([dst, jnp.zeros((pad,), jnp.int32)])
    srcm = srcp.reshape(E_PAD // CHUNK, CHUNK)
    dstm = dstp.reshape(E_PAD // CHUNK, CHUNK)
    dst_eff = _prep(srcm, dstm)
    srcw = srcm.reshape(NW, CPW, CHUNK)
    dstw = dst_eff.reshape(NW, CPW, CHUNK)

    z2_128 = jnp.zeros((RPT, 128), jnp.float32)
    z1 = jnp.zeros((RPT,), jnp.float32)
    ones = jnp.ones((CHUNK,), jnp.float32)

    g0 = _mm(x, W0)
    s0, dg = _agg_deg_128(g0, srcw, dstw, z2_128, z1, ones)
    da = dg[:N].reshape(N, 1)
    db = dg[N1:N1 + N].reshape(N, 1)
    b0r = b0.reshape(1, -1).astype(jnp.float32)
    g1 = _ep_mm(s0[:N], s0[N1:N1 + N], g0, da, db, b0r, W1)
    s1 = _agg_128(g1, srcw, dstw, z2_128)
    b1r = b1.reshape(1, -1).astype(jnp.float32)
    g2 = _ep_mm(s1[:N], s1[N1:N1 + N], g1, da, db, b1r, W2, pad_to=128)
    s2 = _agg_128(g2, srcw, dstw, z2_128)
    b2r = b2.reshape(1, -1).astype(jnp.float32)
    return _ep(s2[:N, :64], s2[N1:N1 + N, :64], g2[:, :64], da, db, b2r)
